# baseline (speedup 1.0000x reference)
import functools

import jax
import jax.numpy as jnp
from jax.experimental import pallas as pl
from jax.experimental.pallas import tpu as pltpu

_VAR_EPS = 1e-8
_BN_EPS = 1e-5
_HALO = 128


def _fused_kernel(x_ref, wm_ref, wv_ref, noise_ref, g_ref, b_ref,
                  o_ref, y_scr, p_ref, p2_ref, sum_scr, sq_scr,
                  scale_scr, shift_scr,
                  *, n_img, kh, kw, c_in, ho_dim, wo_dim):
    hw = ho_dim * wo_dim
    i = pl.program_id(0)

    @pl.when(i < n_img)
    def _compute_phase():
        zpad = jnp.zeros((c_in, _HALO), jnp.bfloat16)
        p = jnp.concatenate(
            [zpad, jnp.maximum(x_ref[0], 0).astype(jnp.bfloat16), zpad],
            axis=1)

        wo = jax.lax.broadcasted_iota(jnp.int32, (1, hw), 1) % wo_dim
        one = jnp.ones((1, hw), jnp.float32)
        zero = jnp.zeros((1, hw), jnp.float32)

        def _mask(cond):
            return jnp.where(cond, one, zero).astype(jnp.bfloat16)

        col_m = {0: _mask(wo >= 1), 1: None, 2: _mask(wo <= wo_dim - 2)}

        for di in range(kh):
            for dj in range(kw):
                t = di * kw + dj
                off = _HALO + (di - 1) * wo_dim + (dj - 1)
                tap = p[:, off:off + hw]
                if col_m[dj] is not None:
                    tap = tap * col_m[dj]
                p_ref[t * c_in:(t + 1) * c_in, :] = tap
                p2_ref[t * c_in:(t + 1) * c_in, :] = tap * tap

        mu = jnp.dot(wm_ref[...], p_ref[...],
                     preferred_element_type=jnp.float32)
        var = jnp.dot(wv_ref[...], p2_ref[...],
                      preferred_element_type=jnp.float32)

        y = mu + jnp.sqrt(_VAR_EPS + var) * noise_ref[0]
        yb = y.astype(jnp.bfloat16)
        y_scr[i] = yb
        y32 = yb.astype(jnp.float32)
        s = jnp.sum(y32, axis=1, keepdims=True)
        sq = jnp.sum(y32 * y32, axis=1, keepdims=True)

        @pl.when(i == 0)
        def _():
            sum_scr[...] = s
            sq_scr[...] = sq

        @pl.when(i > 0)
        def _():
            sum_scr[...] += s
            sq_scr[...] += sq

    @pl.when(i == n_img)
    def _finalize_stats():
        m = n_img * hw
        mean = sum_scr[...] * (1.0 / m)
        var = sq_scr[...] * (1.0 / m) - mean * mean
        inv = g_ref[...] * jax.lax.rsqrt(var + _BN_EPS)
        scale_scr[...] = inv
        shift_scr[...] = b_ref[...] - mean * inv

    @pl.when(i >= n_img)
    def _apply_phase():
        im = i - n_img
        o_ref[0] = (y_scr[im].astype(jnp.float32) * scale_scr[...]
                    + shift_scr[...])


def kernel(x, weight, log_sigma, gamma, beta, noise):
    n, c_in, h, w = x.shape
    c_out, _, kh, kw = weight.shape
    ho, wo = h, w
    hw = ho * wo
    k = c_in * kh * kw

    x3 = x.reshape(n, c_in, hw)
    noise3 = noise.reshape(n, c_out, hw)
    wm = weight.transpose(0, 2, 3, 1).reshape(c_out, k).astype(jnp.bfloat16)
    wv = (0.01 + jnp.exp(2.0 * log_sigma.astype(jnp.float32)))
    wv = wv.transpose(0, 2, 3, 1).reshape(c_out, k).astype(jnp.bfloat16)
    g2 = gamma.reshape(c_out, 1).astype(jnp.float32)
    b2 = beta.reshape(c_out, 1).astype(jnp.float32)

    last = n - 1
    out3 = pl.pallas_call(
        functools.partial(_fused_kernel, n_img=n, kh=kh, kw=kw, c_in=c_in,
                          ho_dim=ho, wo_dim=wo),
        grid=(2 * n,),
        in_specs=[
            pl.BlockSpec((1, c_in, hw),
                         lambda i: (jnp.minimum(i, last), 0, 0)),
            pl.BlockSpec((c_out, k), lambda i: (0, 0)),
            pl.BlockSpec((c_out, k), lambda i: (0, 0)),
            pl.BlockSpec((1, c_out, hw),
                         lambda i: (jnp.minimum(i, last), 0, 0)),
            pl.BlockSpec((c_out, 1), lambda i: (0, 0)),
            pl.BlockSpec((c_out, 1), lambda i: (0, 0)),
        ],
        out_specs=pl.BlockSpec((1, c_out, hw),
                               lambda i: (jnp.maximum(i - (last + 1), 0),
                                          0, 0)),
        out_shape=jax.ShapeDtypeStruct((n, c_out, hw), jnp.float32),
        scratch_shapes=[
            pltpu.VMEM((n, c_out, hw), jnp.bfloat16),
            pltpu.VMEM((k, hw), jnp.bfloat16),
            pltpu.VMEM((k, hw), jnp.bfloat16),
            pltpu.VMEM((c_out, 1), jnp.float32),
            pltpu.VMEM((c_out, 1), jnp.float32),
            pltpu.VMEM((c_out, 1), jnp.float32),
            pltpu.VMEM((c_out, 1), jnp.float32),
        ],
        compiler_params=pltpu.CompilerParams(
            dimension_semantics=("arbitrary",),
            vmem_limit_bytes=100 * 1024 * 1024,
        ),
    )(x3, wm, wv, noise3, g2, b2)

    return out3.reshape(n, c_out, ho, wo)


# speedup vs baseline: 1.0150x; 1.0150x over previous
import functools

import jax
import jax.numpy as jnp
from jax.experimental import pallas as pl
from jax.experimental.pallas import tpu as pltpu

_VAR_EPS = 1e-8
_BN_EPS = 1e-5
_HALO = 128


def _fused_kernel(x_ref, wm_ref, wv_ref, noise_ref, g_ref, b_ref,
                  o_ref, y_scr, p_ref, p2_ref, sum_scr, sq_scr,
                  scale_scr, shift_scr,
                  *, n_img, kh, kw, c_in, ho_dim, wo_dim):
    hw = ho_dim * wo_dim
    i = pl.program_id(0)

    @pl.when(i < n_img)
    def _compute_phase():
        xb = x_ref[0]
        xwin = jnp.concatenate(
            [xb[:, hw - _HALO:], xb, xb[:, :_HALO]], axis=1)
        p = jnp.maximum(xwin, 0).astype(jnp.bfloat16)

        pos = jax.lax.broadcasted_iota(jnp.int32, (1, hw), 1)
        wo = pos % wo_dim
        hos = pos // wo_dim
        one = jnp.ones((1, hw), jnp.float32)
        zero = jnp.zeros((1, hw), jnp.float32)

        def _mask(cond):
            return jnp.where(cond, one, zero).astype(jnp.bfloat16)

        col_m = {0: _mask(wo >= 1), 1: None, 2: _mask(wo <= wo_dim - 2)}
        row_m = {0: _mask(hos >= 1), 1: None, 2: _mask(hos <= ho_dim - 2)}

        for di in range(kh):
            for dj in range(kw):
                t = di * kw + dj
                off = _HALO + (di - 1) * wo_dim + (dj - 1)
                tap = p[:, off:off + hw]
                if row_m[di] is not None and col_m[dj] is not None:
                    tap = tap * (row_m[di] * col_m[dj])
                elif row_m[di] is not None:
                    tap = tap * row_m[di]
                elif col_m[dj] is not None:
                    tap = tap * col_m[dj]
                p_ref[t * c_in:(t + 1) * c_in, :] = tap

        pv = p_ref[...]
        p2_ref[...] = pv * pv
        mu = jnp.dot(wm_ref[...], p_ref[...],
                     preferred_element_type=jnp.float32)
        var = jnp.dot(wv_ref[...], p2_ref[...],
                      preferred_element_type=jnp.float32)

        si = jnp.sqrt(_VAR_EPS + var).astype(jnp.bfloat16)
        yb = mu.astype(jnp.bfloat16) + si * noise_ref[0].astype(jnp.bfloat16)
        y_scr[i] = yb
        y32 = yb.astype(jnp.float32)
        s = jnp.sum(y32, axis=1, keepdims=True)
        sq = jnp.sum(y32 * y32, axis=1, keepdims=True)

        @pl.when(i == 0)
        def _():
            sum_scr[...] = s
            sq_scr[...] = sq

        @pl.when(i > 0)
        def _():
            sum_scr[...] += s
            sq_scr[...] += sq

    @pl.when(i == n_img)
    def _finalize_stats():
        m = n_img * hw
        mean = sum_scr[...] * (1.0 / m)
        var = sq_scr[...] * (1.0 / m) - mean * mean
        inv = g_ref[...] * jax.lax.rsqrt(var + _BN_EPS)
        scale_scr[...] = inv
        shift_scr[...] = b_ref[...] - mean * inv

    @pl.when(i >= n_img)
    def _apply_phase():
        im = i - n_img
        o_ref[0] = (y_scr[im].astype(jnp.float32) * scale_scr[...]
                    + shift_scr[...])


def kernel(x, weight, log_sigma, gamma, beta, noise):
    n, c_in, h, w = x.shape
    c_out, _, kh, kw = weight.shape
    ho, wo = h, w
    hw = ho * wo
    k = c_in * kh * kw

    x3 = x.reshape(n, c_in, hw)
    noise3 = noise.reshape(n, c_out, hw)
    wm = weight.transpose(0, 2, 3, 1).reshape(c_out, k).astype(jnp.bfloat16)
    wv = (0.01 + jnp.exp(2.0 * log_sigma.astype(jnp.float32)))
    wv = wv.transpose(0, 2, 3, 1).reshape(c_out, k).astype(jnp.bfloat16)
    g2 = gamma.reshape(c_out, 1).astype(jnp.float32)
    b2 = beta.reshape(c_out, 1).astype(jnp.float32)

    last = n - 1
    out3 = pl.pallas_call(
        functools.partial(_fused_kernel, n_img=n, kh=kh, kw=kw, c_in=c_in,
                          ho_dim=ho, wo_dim=wo),
        grid=(2 * n,),
        in_specs=[
            pl.BlockSpec((1, c_in, hw),
                         lambda i: (jnp.minimum(i, last), 0, 0)),
            pl.BlockSpec((c_out, k), lambda i: (0, 0)),
            pl.BlockSpec((c_out, k), lambda i: (0, 0)),
            pl.BlockSpec((1, c_out, hw),
                         lambda i: (jnp.minimum(i, last), 0, 0)),
            pl.BlockSpec((c_out, 1), lambda i: (0, 0)),
            pl.BlockSpec((c_out, 1), lambda i: (0, 0)),
        ],
        out_specs=pl.BlockSpec((1, c_out, hw),
                               lambda i: (jnp.maximum(i - (last + 1), 0),
                                          0, 0)),
        out_shape=jax.ShapeDtypeStruct((n, c_out, hw), jnp.float32),
        scratch_shapes=[
            pltpu.VMEM((n, c_out, hw), jnp.bfloat16),
            pltpu.VMEM((k, hw), jnp.bfloat16),
            pltpu.VMEM((k, hw), jnp.bfloat16),
            pltpu.VMEM((c_out, 1), jnp.float32),
            pltpu.VMEM((c_out, 1), jnp.float32),
            pltpu.VMEM((c_out, 1), jnp.float32),
            pltpu.VMEM((c_out, 1), jnp.float32),
        ],
        compiler_params=pltpu.CompilerParams(
            dimension_semantics=("arbitrary",),
            vmem_limit_bytes=100 * 1024 * 1024,
        ),
    )(x3, wm, wv, noise3, g2, b2)

    return out3.reshape(n, c_out, ho, wo)


# speedup vs baseline: 1.0547x; 1.0391x over previous
import functools

import jax
import jax.numpy as jnp
from jax.experimental import pallas as pl
from jax.experimental.pallas import tpu as pltpu

_VAR_EPS = 1e-8
_BN_EPS = 1e-5
_HALO = 128


def _fused_kernel(x_ref, wm_ref, wv_ref, noise_ref, g_ref, b_ref,
                  o_ref, y_scr, p_ref, p2_ref, sum_scr, sq_scr,
                  scale_scr, shift_scr,
                  *, n_img, kh, kw, c_in, ho_dim, wo_dim):
    hw = ho_dim * wo_dim
    i = pl.program_id(0)

    @pl.when(i < n_img)
    def _compute_phase():
        xb = x_ref[0]
        xwin = jnp.concatenate(
            [xb[:, hw - _HALO:], xb, xb[:, :_HALO]], axis=1)
        p = jnp.maximum(xwin, 0).astype(jnp.bfloat16)

        pos = jax.lax.broadcasted_iota(jnp.int32, (1, hw), 1)
        wo = pos % wo_dim
        hos = pos // wo_dim
        one = jnp.ones((1, hw), jnp.float32)
        zero = jnp.zeros((1, hw), jnp.float32)

        def _mask(cond):
            return jnp.where(cond, one, zero).astype(jnp.bfloat16)

        col_m = {0: _mask(wo >= 1), 1: None, 2: _mask(wo <= wo_dim - 2)}
        row_m = {0: _mask(hos >= 1), 1: None, 2: _mask(hos <= ho_dim - 2)}

        for di in range(kh):
            for dj in range(kw):
                t = di * kw + dj
                off = _HALO + (di - 1) * wo_dim + (dj - 1)
                tap = p[:, off:off + hw]
                if row_m[di] is not None and col_m[dj] is not None:
                    tap = tap * (row_m[di] * col_m[dj])
                elif row_m[di] is not None:
                    tap = tap * row_m[di]
                elif col_m[dj] is not None:
                    tap = tap * col_m[dj]
                p_ref[t * c_in:(t + 1) * c_in, :] = tap

        pv = p_ref[...]
        p2_ref[...] = pv * pv
        mu = jnp.dot(wm_ref[...], p_ref[...],
                     preferred_element_type=jnp.float32)
        var = jnp.dot(wv_ref[...], p2_ref[...],
                      preferred_element_type=jnp.float32)

        y = mu + jnp.sqrt(_VAR_EPS + var) * noise_ref[0]
        yb = y.astype(jnp.bfloat16)
        y_scr[i] = yb
        y32 = yb.astype(jnp.float32)
        s = jnp.sum(y32, axis=1, keepdims=True)
        sq = jnp.sum(y32 * y32, axis=1, keepdims=True)

        @pl.when(i == 0)
        def _():
            sum_scr[...] = s
            sq_scr[...] = sq

        @pl.when(i > 0)
        def _():
            sum_scr[...] += s
            sq_scr[...] += sq

    @pl.when(i == n_img)
    def _finalize_stats():
        m = n_img * hw
        mean = sum_scr[...] * (1.0 / m)
        var = sq_scr[...] * (1.0 / m) - mean * mean
        inv = g_ref[...] * jax.lax.rsqrt(var + _BN_EPS)
        scale_scr[...] = inv
        shift_scr[...] = b_ref[...] - mean * inv

    @pl.when(i >= n_img)
    def _apply_phase():
        im = i - n_img
        o_ref[0] = (y_scr[im].astype(jnp.float32) * scale_scr[...]
                    + shift_scr[...])


def kernel(x, weight, log_sigma, gamma, beta, noise):
    n, c_in, h, w = x.shape
    c_out, _, kh, kw = weight.shape
    ho, wo = h, w
    hw = ho * wo
    k = c_in * kh * kw

    x3 = x.reshape(n, c_in, hw)
    noise3 = noise.reshape(n, c_out, hw)
    wm = weight.transpose(0, 2, 3, 1).reshape(c_out, k).astype(jnp.bfloat16)
    wv = (0.01 + jnp.exp(2.0 * log_sigma.astype(jnp.float32)))
    wv = wv.transpose(0, 2, 3, 1).reshape(c_out, k).astype(jnp.bfloat16)
    g2 = gamma.reshape(c_out, 1).astype(jnp.float32)
    b2 = beta.reshape(c_out, 1).astype(jnp.float32)

    last = n - 1
    out3 = pl.pallas_call(
        functools.partial(_fused_kernel, n_img=n, kh=kh, kw=kw, c_in=c_in,
                          ho_dim=ho, wo_dim=wo),
        grid=(2 * n,),
        in_specs=[
            pl.BlockSpec((1, c_in, hw),
                         lambda i: (jnp.minimum(i, last), 0, 0)),
            pl.BlockSpec((c_out, k), lambda i: (0, 0)),
            pl.BlockSpec((c_out, k), lambda i: (0, 0)),
            pl.BlockSpec((1, c_out, hw),
                         lambda i: (jnp.minimum(i, last), 0, 0)),
            pl.BlockSpec((c_out, 1), lambda i: (0, 0)),
            pl.BlockSpec((c_out, 1), lambda i: (0, 0)),
        ],
        out_specs=pl.BlockSpec((1, c_out, hw),
                               lambda i: (jnp.maximum(i - (last + 1), 0),
                                          0, 0)),
        out_shape=jax.ShapeDtypeStruct((n, c_out, hw), jnp.float32),
        scratch_shapes=[
            pltpu.VMEM((n, c_out, hw), jnp.bfloat16),
            pltpu.VMEM((k, hw), jnp.bfloat16),
            pltpu.VMEM((k, hw), jnp.bfloat16),
            pltpu.VMEM((c_out, 1), jnp.float32),
            pltpu.VMEM((c_out, 1), jnp.float32),
            pltpu.VMEM((c_out, 1), jnp.float32),
            pltpu.VMEM((c_out, 1), jnp.float32),
        ],
        compiler_params=pltpu.CompilerParams(
            dimension_semantics=("arbitrary",),
            vmem_limit_bytes=100 * 1024 * 1024,
        ),
    )(x3, wm, wv, noise3, g2, b2)

    return out3.reshape(n, c_out, ho, wo)
